# baseline (speedup 1.0000x reference)
import jax
import jax.numpy as jnp
from jax import lax
from jax.experimental import pallas as pl
from jax.experimental.pallas import tpu as pltpu
from jax.experimental.pallas import tpu_sc as plsc

EMB = 64
_NC = 2
_NS = 16
NW = _NC * _NS
G = 128
R = 2


def _emb_body(xt_hbm, sum_hbm, out_hbm, idx_v, gbufs, obufs, sems_g, sems_o):
    ns = xt_hbm.shape[0]
    wid = lax.axis_index("s") * _NC + lax.axis_index("c")
    pltpu.sync_copy(xt_hbm.at[:, wid], idx_v)
    iota = lax.iota(jnp.int32, 16)

    def fire(s, k):
        pltpu.async_copy(sum_hbm.at[idx_v.at[s]], gbufs[k], sems_g[k])

    def wait_gather(k):
        pltpu.make_async_copy(sum_hbm.at[idx_v.at[0]], gbufs[k], sems_g[k]).wait()

    def out_dst(s):
        return out_hbm.at[s, :, pl.ds(wid * G, G)]

    def drain_out(k):
        pltpu.make_async_copy(obufs[k], out_dst(0), sems_o[k]).wait()

    for k in range(R):
        fire(k, k)

    @pl.loop(0, ns, step=R)
    def _pair(s0):
        for k in range(R):
            s = s0 + k
            wait_gather(k)

            @pl.loop(0, G // 16)
            def _blk(jb):
                rows = jb * 16 + iota
                for c in range(EMB):
                    left = plsc.load_gather(gbufs[k], [rows, jnp.full((16,), c, jnp.int32)])
                    right = plsc.load_gather(gbufs[k], [rows, jnp.full((16,), EMB + c, jnp.int32)])
                    obufs[k][c, pl.ds(jb * 16, 16)] = left + right

            @pl.when(s + R < ns)
            def _():
                fire(s + R, k)

            @pl.when(s >= R)
            def _():
                drain_out(k)

            pltpu.async_copy(obufs[k], out_dst(s), sems_o[k])

    for k in range(R):
        drain_out(k)


def kernel(x, word_table, pe_table):
    b, s = x.shape
    sum128 = jnp.pad(word_table + pe_table, ((0, 0), (0, EMB)))
    xt3 = jnp.transpose(x).reshape(s, b // G, G)
    mesh = plsc.VectorSubcoreMesh(core_axis_name="c", subcore_axis_name="s")
    out_t = pl.kernel(
        _emb_body,
        out_type=jax.ShapeDtypeStruct((s, EMB, b), jnp.float32),
        mesh=mesh,
        compiler_params=pltpu.CompilerParams(needs_layout_passes=False),
        scratch_types=[
            pltpu.VMEM((s, G), jnp.int32),
            [pltpu.VMEM((G, 2 * EMB), jnp.float32) for _ in range(R)],
            [pltpu.VMEM((EMB, G), jnp.float32) for _ in range(R)],
            [pltpu.SemaphoreType.DMA for _ in range(R)],
            [pltpu.SemaphoreType.DMA for _ in range(R)],
        ],
    )(xt3, sum128)
    return jnp.transpose(out_t, (2, 0, 1))


# speedup vs baseline: 1.5730x; 1.5730x over previous
import jax
import jax.numpy as jnp
from jax import lax
from jax.experimental import pallas as pl
from jax.experimental.pallas import tpu as pltpu
from jax.experimental.pallas import tpu_sc as plsc

EMB = 64
_NC = 2
_NS = 16
NW = _NC * _NS
G = 128
R = 2


def _emb_body(xt_hbm, sum_hbm, out_hbm, idx_v, gbufs, obufs, sems_g, sems_o):
    ns = xt_hbm.shape[0]
    wid = lax.axis_index("s") * _NC + lax.axis_index("c")
    pltpu.sync_copy(xt_hbm.at[:, wid], idx_v)
    iota = lax.iota(jnp.int32, 16)

    def fire(s, k):
        pltpu.async_copy(sum_hbm.at[idx_v.at[s]], gbufs[k], sems_g[k])

    def wait_gather(k):
        pltpu.make_async_copy(sum_hbm.at[idx_v.at[0]], gbufs[k], sems_g[k]).wait()

    def out_dst(s):
        return out_hbm.at[s, :, pl.ds(wid * G, G)]

    def drain_out(k):
        pltpu.make_async_copy(obufs[k], out_dst(0), sems_o[k]).wait()

    for k in range(R):
        fire(k, k)

    @pl.loop(0, ns, step=R)
    def _pair(s0):
        for k in range(R):
            s = s0 + k
            wait_gather(k)

            @pl.loop(0, G, unroll=2)
            def _row(j):
                jv = jnp.full((16,), 0, jnp.int32) + j
                for c in range(EMB // 16):
                    s_ = pl.ds(c * 16, 16)
                    val = gbufs[k][j, s_] + gbufs[k][j, pl.ds(EMB + c * 16, 16)]
                    plsc.store_scatter(obufs[k], [c * 16 + iota, jv], val)

            @pl.when(s + R < ns)
            def _():
                fire(s + R, k)

            @pl.when(s >= R)
            def _():
                drain_out(k)

            pltpu.async_copy(obufs[k], out_dst(s), sems_o[k])

    for k in range(R):
        drain_out(k)


def kernel(x, word_table, pe_table):
    b, s = x.shape
    sum128 = jnp.pad(word_table + pe_table, ((0, 0), (0, EMB)))
    xt3 = jnp.transpose(x).reshape(s, b // G, G)
    mesh = plsc.VectorSubcoreMesh(core_axis_name="c", subcore_axis_name="s")
    out_t = pl.kernel(
        _emb_body,
        out_type=jax.ShapeDtypeStruct((s, EMB, b), jnp.float32),
        mesh=mesh,
        compiler_params=pltpu.CompilerParams(needs_layout_passes=False),
        scratch_types=[
            pltpu.VMEM((s, G), jnp.int32),
            [pltpu.VMEM((G, 2 * EMB), jnp.float32) for _ in range(R)],
            [pltpu.VMEM((EMB, G), jnp.float32) for _ in range(R)],
            [pltpu.SemaphoreType.DMA for _ in range(R)],
            [pltpu.SemaphoreType.DMA for _ in range(R)],
        ],
    )(xt3, sum128)
    return jnp.transpose(out_t, (2, 0, 1))


# speedup vs baseline: 2.3184x; 1.4739x over previous
import jax
import jax.numpy as jnp
from jax import lax
from jax.experimental import pallas as pl
from jax.experimental.pallas import tpu as pltpu
from jax.experimental.pallas import tpu_sc as plsc

EMB = 64
_NC = 2
_NS = 16
NW = _NC * _NS
G = 128
R = 2


def _emb_body(x_hbm, sum_hbm, out_hbm, idx_v, gbufs, obufs, sems_g, sems_o):
    ng = x_hbm.shape[0] // NW
    wid = lax.axis_index("s") * _NC + lax.axis_index("c")
    pltpu.sync_copy(x_hbm.at[pl.ds(wid * ng, ng)], idx_v)
    base = wid * ng * G

    def fire(g, k):
        pltpu.async_copy(sum_hbm.at[idx_v.at[g]], gbufs[k], sems_g[k])

    def wait_gather(k):
        pltpu.make_async_copy(sum_hbm.at[idx_v.at[0]], gbufs[k], sems_g[k]).wait()

    def drain_out(k):
        pltpu.make_async_copy(obufs[k], out_hbm.at[pl.ds(base, G)], sems_o[k]).wait()

    for k in range(R):
        fire(k, k)

    @pl.loop(0, ng, step=R)
    def _pair(g):
        for k in range(R):
            gi = g + k
            wait_gather(k)

            @pl.loop(0, G, unroll=4)
            def _row(j):
                for c in range(EMB // 16):
                    s = pl.ds(c * 16, 16)
                    obufs[k][j, s] = gbufs[k][j, s]

            @pl.when(gi + R < ng)
            def _():
                fire(gi + R, k)

            @pl.when(gi >= R)
            def _():
                drain_out(k)

            pltpu.async_copy(obufs[k], out_hbm.at[pl.ds(base + gi * G, G)], sems_o[k])

    for k in range(R):
        drain_out(k)


def kernel(x, word_table, pe_table):
    b, s = x.shape
    n = b * s
    xg = x.reshape(n // G, G)
    sum128 = jnp.pad(word_table + pe_table, ((0, 0), (0, EMB)))
    mesh = plsc.VectorSubcoreMesh(core_axis_name="c", subcore_axis_name="s")
    out = pl.kernel(
        _emb_body,
        out_type=jax.ShapeDtypeStruct((n, EMB), jnp.float32),
        mesh=mesh,
        scratch_types=[
            pltpu.VMEM((n // G // NW, G), jnp.int32),
            [pltpu.VMEM((G, 2 * EMB), jnp.float32) for _ in range(R)],
            [pltpu.VMEM((G, EMB), jnp.float32) for _ in range(R)],
            [pltpu.SemaphoreType.DMA for _ in range(R)],
            [pltpu.SemaphoreType.DMA for _ in range(R)],
        ],
    )(xg, sum128)
    return out.reshape(b, s, EMB)
